# baseline (speedup 1.0000x reference)
import functools

import jax
import jax.numpy as jnp
from jax import lax
from jax.experimental import pallas as pl
from jax.experimental.pallas import tpu as pltpu
from jax.experimental.pallas import tpu_sc as plsc

_IW = 128
_G = 5


@functools.partial(jax.jit, static_argnums=(2, 3))
def _gather_rows(idx_grp, table, n_groups, d):
    mesh = plsc.VectorSubcoreMesh(core_axis_name="c", subcore_axis_name="s")
    info = plsc.get_sparse_core_info()
    nc, ns = info.num_cores, info.num_subcores
    nw = nc * ns
    grp_per_w = n_groups // nw
    n_chunks = grp_per_w // _G
    assert n_groups % nw == 0 and grp_per_w % _G == 0 and n_chunks % 2 == 0
    half = n_chunks // 2

    @functools.partial(
        pl.kernel,
        mesh=mesh,
        compiler_params=pltpu.CompilerParams(use_tc_tiling_on_sc=False),
        out_type=jax.ShapeDtypeStruct((n_groups, _IW, d), jnp.float32),
        scratch_types=[
            pltpu.VMEM((grp_per_w, _IW), jnp.int32),
            pltpu.VMEM((2, _G, _IW, d), jnp.float32),
            pltpu.SemaphoreType.DMA,
            pltpu.SemaphoreType.DMA,
        ],
    )
    def k(idx_hbm, table_hbm, out_hbm, idx_v, rows_v, gsem, osem):
        wid = lax.axis_index("s") * nc + lax.axis_index("c")
        base = wid * grp_per_w
        pltpu.sync_copy(idx_hbm.at[pl.ds(base, grp_per_w)], idx_v)

        def gather_chunk(c, slot):
            for j in range(_G):
                pltpu.async_copy(
                    table_hbm.at[idx_v.at[c * _G + j]], rows_v.at[slot, j], gsem
                )

        def wait_gathers(slot):
            pltpu.make_async_copy(
                out_hbm.at[pl.ds(0, _G)], rows_v.at[slot], gsem
            ).wait()

        def start_out(c, slot):
            pltpu.async_copy(
                rows_v.at[slot], out_hbm.at[pl.ds(base + c * _G, _G)], osem
            )

        def wait_out(slot):
            pltpu.make_async_copy(
                rows_v.at[slot], out_hbm.at[pl.ds(0, _G)], osem
            ).wait()

        gather_chunk(0, 0)

        def body(t, carry):
            c0 = 2 * t

            @pl.when(t > 0)
            def _():
                wait_out(1)

            gather_chunk(c0 + 1, 1)
            wait_gathers(0)
            start_out(c0, 0)

            wait_out(0)

            @pl.when(t < half - 1)
            def _():
                gather_chunk(c0 + 2, 0)

            wait_gathers(1)
            start_out(c0 + 1, 1)
            return carry

        lax.fori_loop(0, half, body, 0)
        wait_out(1)

    return k(idx_grp, table)


def kernel(sequence, table):
    b, l = sequence.shape
    v, d = table.shape
    n = b * l
    idx_grp = jnp.transpose(sequence).reshape(n // _IW, _IW)
    out = _gather_rows(idx_grp, table, n // _IW, d)
    return out.reshape(l, b, d)


# speedup vs baseline: 1.8376x; 1.8376x over previous
import functools

import jax
import jax.numpy as jnp
from jax import lax
from jax.experimental import pallas as pl
from jax.experimental.pallas import tpu as pltpu
from jax.experimental.pallas import tpu_sc as plsc

_IW = 128
_G = 5


@functools.partial(jax.jit, static_argnums=(2, 3))
def _gather_rows(idx_grp, table, n_groups, d):
    mesh = plsc.VectorSubcoreMesh(core_axis_name="c", subcore_axis_name="s")
    info = plsc.get_sparse_core_info()
    nc, ns = info.num_cores, info.num_subcores
    nw = nc * ns
    grp_per_w = n_groups // nw
    n_chunks = grp_per_w // _G
    assert n_groups % nw == 0 and grp_per_w % _G == 0 and n_chunks % 2 == 0
    half = n_chunks // 2

    @functools.partial(
        pl.kernel,
        mesh=mesh,
        compiler_params=pltpu.CompilerParams(use_tc_tiling_on_sc=False),
        out_type=jax.ShapeDtypeStruct((n_groups, _IW, 2 * d), jnp.float32),
        scratch_types=[
            pltpu.VMEM((grp_per_w, _IW), jnp.int32),
            pltpu.VMEM((2, _G, _IW, d), jnp.float32),
            pltpu.SemaphoreType.DMA,
            pltpu.SemaphoreType.DMA,
        ],
    )
    def k(idx_hbm, table_hbm, out_hbm, idx_v, rows_v, gsem, osem):
        wid = lax.axis_index("s") * nc + lax.axis_index("c")
        base = wid * grp_per_w
        pltpu.sync_copy(idx_hbm.at[pl.ds(base, grp_per_w)], idx_v)

        def gather_chunk(c, slot):
            for j in range(_G):
                pltpu.async_copy(
                    table_hbm.at[idx_v.at[c * _G + j]], rows_v.at[slot, j], gsem
                )

        def wait_gathers(slot):
            pltpu.make_async_copy(
                out_hbm.at[pl.ds(0, _G), :, pl.ds(0, d)], rows_v.at[slot], gsem
            ).wait()

        def start_out(c, slot):
            pltpu.async_copy(
                rows_v.at[slot],
                out_hbm.at[pl.ds(base + c * _G, _G), :, pl.ds(0, d)],
                osem,
            )

        def wait_out(slot):
            pltpu.make_async_copy(
                rows_v.at[slot], out_hbm.at[pl.ds(0, _G), :, pl.ds(0, d)], osem
            ).wait()

        gather_chunk(0, 0)

        def body(t, carry):
            c0 = 2 * t

            @pl.when(t > 0)
            def _():
                wait_out(1)

            gather_chunk(c0 + 1, 1)
            wait_gathers(0)
            start_out(c0, 0)

            wait_out(0)

            @pl.when(t < half - 1)
            def _():
                gather_chunk(c0 + 2, 0)

            wait_gathers(1)
            start_out(c0 + 1, 1)
            return carry

        lax.fori_loop(0, half, body, 0)
        wait_out(1)

    return k(idx_grp, table)


def kernel(sequence, table):
    b, l = sequence.shape
    v, d = table.shape
    n = b * l
    idx_grp = jnp.transpose(sequence).reshape(n // _IW, _IW)
    padded = _gather_rows(idx_grp, table, n // _IW, d)
    return padded[:, :, :d].reshape(l, b, d)
